# baseline (speedup 1.0000x reference)
import jax
import jax.numpy as jnp
from jax import lax
from jax.experimental import pallas as pl
from jax.experimental.pallas import tpu as pltpu
from jax.experimental.pallas import tpu_sc as plsc

_BT = 16
_L = 16


def _sc_body(x_ref, pe_ref, o_ref):
    @pl.loop(0, _BT)
    def _(r):
        @pl.loop(0, 1024, step=_L)
        def _(c):
            o_ref[0, r, pl.ds(c, _L)] = (
                x_ref[0, r, pl.ds(c, _L)] + pe_ref[r, pl.ds(c, _L)]
            )


def kernel(x, pos_embed):
    B, T, C = x.shape
    pe = pos_embed[:T]
    mesh = plsc.VectorSubcoreMesh(core_axis_name="c", subcore_axis_name="s")

    @pl.kernel(out_type=jax.ShapeDtypeStruct((B, T, C), x.dtype), mesh=mesh)
    def sc_add(x_hbm, pe_hbm, o_hbm):
        pltpu.emit_pipeline(
            _sc_body,
            grid=(T // _BT, B),
            in_specs=[
                pl.BlockSpec((1, _BT, C), lambda i, b: (b, i, 0)),
                pl.BlockSpec((_BT, C), lambda i, b: (i, 0)),
            ],
            out_specs=[pl.BlockSpec((1, _BT, C), lambda i, b: (b, i, 0))],
            core_axis_name=("c", "s"),
            dimension_semantics=(pltpu.PARALLEL, pltpu.ARBITRARY),
        )(x_hbm, pe_hbm, o_hbm)

    return sc_add(x, pe)


# speedup vs baseline: 2.9181x; 2.9181x over previous
import jax
import jax.numpy as jnp
from jax import lax
from jax.experimental import pallas as pl
from jax.experimental.pallas import tpu as pltpu
from jax.experimental.pallas import tpu_sc as plsc

_BT = 4
_L = 16


def _make_sc_body(B, C):
    def _sc_body(x_ref, pe_ref, o_ref):
        @pl.loop(0, _BT)
        def _(r):
            @plsc.parallel_loop(0, C, step=_L, unroll=4)
            def _(c):
                pe_v = pe_ref[r, pl.ds(c, _L)]
                for b in range(B):
                    o_ref[b, r, pl.ds(c, _L)] = x_ref[b, r, pl.ds(c, _L)] + pe_v

    return _sc_body


def kernel(x, pos_embed):
    B, T, C = x.shape
    pe = pos_embed[:T]
    mesh = plsc.VectorSubcoreMesh(core_axis_name="c", subcore_axis_name="s")

    @pl.kernel(out_type=jax.ShapeDtypeStruct((B, T, C), x.dtype), mesh=mesh)
    def sc_add(x_hbm, pe_hbm, o_hbm):
        pltpu.emit_pipeline(
            _make_sc_body(B, C),
            grid=(T // _BT,),
            in_specs=[
                pl.BlockSpec((B, _BT, C), lambda i: (0, i, 0)),
                pl.BlockSpec((_BT, C), lambda i: (i, 0)),
            ],
            out_specs=[pl.BlockSpec((B, _BT, C), lambda i: (0, i, 0))],
            core_axis_name=("c", "s"),
            dimension_semantics=(pltpu.PARALLEL,),
        )(x_hbm, pe_hbm, o_hbm)

    return sc_add(x, pe)


# speedup vs baseline: 2.9302x; 1.0041x over previous
import jax
import jax.numpy as jnp
from jax import lax
from jax.experimental import pallas as pl
from jax.experimental.pallas import tpu as pltpu
from jax.experimental.pallas import tpu_sc as plsc

_BT = 4
_L = 16


def _make_sc_body(B, C):
    def _sc_body(x_ref, pe_ref, o_ref):
        @pl.loop(0, _BT)
        def _(r):
            @plsc.parallel_loop(0, C, step=_L, unroll=8)
            def _(c):
                pe_v = pe_ref[r, pl.ds(c, _L)]
                for b in range(B):
                    o_ref[b, r, pl.ds(c, _L)] = x_ref[b, r, pl.ds(c, _L)] + pe_v

    return _sc_body


def kernel(x, pos_embed):
    B, T, C = x.shape
    pe = pos_embed[:T]
    mesh = plsc.VectorSubcoreMesh(core_axis_name="c", subcore_axis_name="s")

    @pl.kernel(out_type=jax.ShapeDtypeStruct((B, T, C), x.dtype), mesh=mesh)
    def sc_add(x_hbm, pe_hbm, o_hbm):
        pltpu.emit_pipeline(
            _make_sc_body(B, C),
            grid=(T // _BT,),
            in_specs=[
                pl.BlockSpec((B, _BT, C), lambda i: (0, i, 0)),
                pl.BlockSpec((_BT, C), lambda i: (i, 0)),
            ],
            out_specs=[pl.BlockSpec((B, _BT, C), lambda i: (0, i, 0))],
            core_axis_name=("c", "s"),
            dimension_semantics=(pltpu.PARALLEL,),
        )(x_hbm, pe_hbm, o_hbm)

    return sc_add(x, pe)


# speedup vs baseline: 4.4737x; 1.5268x over previous
import jax
import jax.numpy as jnp
from jax.experimental import pallas as pl
from jax.experimental.pallas import tpu as pltpu

_BT = 2048


def _add_body(x_ref, pe_ref, o_ref):
    o_ref[...] = x_ref[...] + pe_ref[...]


def kernel(x, pos_embed):
    B, T, C = x.shape
    pe = pos_embed[:T]
    grid = (T // _BT, B)
    return pl.pallas_call(
        _add_body,
        grid=grid,
        in_specs=[
            pl.BlockSpec((1, _BT, C), lambda t, b: (b, t, 0)),
            pl.BlockSpec((_BT, C), lambda t, b: (t, 0)),
        ],
        out_specs=pl.BlockSpec((1, _BT, C), lambda t, b: (b, t, 0)),
        out_shape=jax.ShapeDtypeStruct((B, T, C), x.dtype),
        compiler_params=pltpu.CompilerParams(
            dimension_semantics=("arbitrary", "arbitrary"),
        ),
    )(x, pe)
